# baseline (speedup 1.0000x reference)
import functools

import jax
import jax.numpy as jnp
import numpy as np
from jax.experimental import pallas as pl
from jax.sharding import Mesh, PartitionSpec as P

K = 16


def _knn_block_kernel(x_ref, xloc_ref, out_ref, *, blk_rows: int, n: int):
    xall = x_ref[0]
    xrows = xloc_ref[0]
    inner = jax.lax.dot_general(
        xall, xrows, (((0,), (0,)), ((), ())),
        preferred_element_type=jnp.float32,
        precision=jax.lax.Precision.DEFAULT)
    xsq = xall * xall
    ones = jnp.ones((xall.shape[0], 1), dtype=jnp.float32)
    sq_col = jax.lax.dot_general(
        xsq, ones, (((0,), (0,)), ((), ())),
        preferred_element_type=jnp.float32,
        precision=jax.lax.Precision.HIGHEST)
    s = sq_col - 2.0 * inner
    iota = jax.lax.broadcasted_iota(
        jnp.int32, (n, blk_rows), 0).astype(jnp.float32)
    for k in range(K):
        m = jnp.min(s, axis=0, keepdims=True)
        hit = s == m
        idx = jnp.min(jnp.where(hit, iota, jnp.float32(n)),
                      axis=0, keepdims=True)
        out_ref[0, k, :] = idx[0].astype(jnp.int32)
        if k < K - 1:
            s = jnp.where(iota == idx, jnp.inf, s)


def _knn_rows(xs, xloc, *, blk_rows, n):
    B, D, N = xs.shape
    NL = xloc.shape[2]
    grid = (B, NL // blk_rows)
    return pl.pallas_call(
        functools.partial(_knn_block_kernel, blk_rows=blk_rows, n=N),
        grid=grid,
        in_specs=[
            pl.BlockSpec((1, D, N), lambda b, i: (b, 0, 0)),
            pl.BlockSpec((1, D, blk_rows), lambda b, i: (b, 0, i)),
        ],
        out_specs=pl.BlockSpec((1, K, blk_rows), lambda b, i: (b, 0, i)),
        out_shape=jax.ShapeDtypeStruct((B, K, NL), jnp.int32),
    )(xs, xloc)


def kernel(x):
    B, D, N, _ = x.shape
    xs = x.reshape(B, D, N)
    blk_rows = 1024
    devs = jax.devices()
    ndev = 2 if len(devs) >= 2 and N % 2 == 0 else 1
    if ndev > 1:
        mesh = Mesh(np.array(devs[:ndev]), ("d",))

        def shard_fn(xs_rep):
            nl = N // ndev
            row0 = jax.lax.axis_index("d") * nl
            xloc = jax.lax.dynamic_slice(xs_rep, (0, 0, row0), (B, D, nl))
            nn = _knn_rows(xs_rep, xloc, blk_rows=blk_rows, n=N)
            nn_idx = jnp.swapaxes(nn, 1, 2)
            center = row0 + jax.lax.broadcasted_iota(jnp.int32, (B, nl, K), 1)
            return jnp.stack((nn_idx, center), axis=0)

        return jax.shard_map(
            shard_fn, mesh=mesh, in_specs=P(),
            out_specs=P(None, None, "d", None),
            check_vma=False,
        )(xs)
    nn = _knn_rows(xs, xs, blk_rows=blk_rows, n=N)
    nn_idx = jnp.swapaxes(nn, 1, 2)
    center_idx = jnp.broadcast_to(
        jnp.arange(N, dtype=nn_idx.dtype)[None, :, None], (B, N, K))
    return jnp.stack((nn_idx, center_idx), axis=0)


# speedup vs baseline: 1.0693x; 1.0693x over previous
import functools

import jax
import jax.numpy as jnp
import numpy as np
from jax.experimental import pallas as pl
from jax.sharding import Mesh, PartitionSpec as P

K = 16


def _knn_block_kernel(x_ref, xloc_ref, out_ref, *, blk_rows: int, n: int):
    xall = x_ref[0]
    xrows = xloc_ref[0]
    inner = jax.lax.dot_general(
        xall, xrows, (((0,), (0,)), ((), ())),
        preferred_element_type=jnp.float32,
        precision=jax.lax.Precision.DEFAULT)
    xsq = xall * xall
    ones = jnp.ones((xall.shape[0], 1), dtype=jnp.float32)
    sq_col = jax.lax.dot_general(
        xsq, ones, (((0,), (0,)), ((), ())),
        preferred_element_type=jnp.float32,
        precision=jax.lax.Precision.HIGHEST)
    s = sq_col - 2.0 * inner
    iota = jax.lax.broadcasted_iota(
        jnp.int32, (n, blk_rows), 0).astype(jnp.float32)
    for k in range(K):
        m = jnp.min(s, axis=0, keepdims=True)
        hit = s == m
        idx = jnp.min(jnp.where(hit, iota, jnp.float32(n)),
                      axis=0, keepdims=True)
        out_ref[0, k, :] = idx[0].astype(jnp.int32)
        if k < K - 1:
            s = jnp.where(iota == idx, jnp.inf, s)


def _knn_rows(xs, xloc, *, blk_rows, n):
    B, D, N = xs.shape
    NL = xloc.shape[2]
    grid = (B, NL // blk_rows)
    return pl.pallas_call(
        functools.partial(_knn_block_kernel, blk_rows=blk_rows, n=N),
        grid=grid,
        in_specs=[
            pl.BlockSpec((1, D, N), lambda b, i: (b, 0, 0)),
            pl.BlockSpec((1, D, blk_rows), lambda b, i: (b, 0, i)),
        ],
        out_specs=pl.BlockSpec((1, K, blk_rows), lambda b, i: (b, 0, i)),
        out_shape=jax.ShapeDtypeStruct((B, K, NL), jnp.int32),
    )(xs, xloc)


def kernel(x):
    B, D, N, _ = x.shape
    xs = x.reshape(B, D, N)
    blk_rows = 512
    devs = jax.devices()
    ndev = 2 if len(devs) >= 2 and N % 2 == 0 else 1
    if ndev > 1:
        mesh = Mesh(np.array(devs[:ndev]), ("d",))

        def shard_fn(xs_rep):
            nl = N // ndev
            row0 = jax.lax.axis_index("d") * nl
            xloc = jax.lax.dynamic_slice(xs_rep, (0, 0, row0), (B, D, nl))
            nn = _knn_rows(xs_rep, xloc, blk_rows=blk_rows, n=N)
            nn_idx = jnp.swapaxes(nn, 1, 2)
            center = row0 + jax.lax.broadcasted_iota(jnp.int32, (B, nl, K), 1)
            return jnp.stack((nn_idx, center), axis=0)

        return jax.shard_map(
            shard_fn, mesh=mesh, in_specs=P(),
            out_specs=P(None, None, "d", None),
            check_vma=False,
        )(xs)
    nn = _knn_rows(xs, xs, blk_rows=blk_rows, n=N)
    nn_idx = jnp.swapaxes(nn, 1, 2)
    center_idx = jnp.broadcast_to(
        jnp.arange(N, dtype=nn_idx.dtype)[None, :, None], (B, N, K))
    return jnp.stack((nn_idx, center_idx), axis=0)
